# baseline (speedup 1.0000x reference)
import functools

import jax
import jax.numpy as jnp
from jax import lax
from jax.experimental import pallas as pl
from jax.experimental.pallas import tpu as pltpu
from jax.experimental.pallas import tpu_sc as plsc

N = 10000
IN_DIM = 128
HID_DIM = 64

NC = 2
NS = 16
NW = NC * NS
L = 16

CHUNK = 128
N_ACC = 10112
ROWS_PER_TILE = N_ACC // NS


def _flat_tile_id():
    return lax.axis_index("c") * NS + lax.axis_index("s")


def _deg_body(nch16, dst_hbm, out_hbm, dst_v, acc):
    wid = _flat_tile_id()
    pltpu.sync_copy(dst_hbm.at[wid], dst_v)

    zero16 = jnp.zeros((L,), jnp.float32)

    @pl.loop(0, N_ACC // L)
    def _(i):
        acc[pl.ds(i * L, L)] = zero16

    ones16 = jnp.ones((L,), jnp.float32)

    @pl.loop(0, nch16)
    def _(k):
        idx = dst_v[k]
        plsc.addupdate_scatter(acc, [idx], ones16)

    pltpu.sync_copy(acc, out_hbm.at[wid])


def _make_deg_call(nch16):
    mesh = plsc.VectorSubcoreMesh(core_axis_name="c", subcore_axis_name="s")
    return pl.kernel(
        functools.partial(_deg_body, nch16),
        out_type=jax.ShapeDtypeStruct((NW, N_ACC), jnp.float32),
        mesh=mesh,
        scratch_types=[
            pltpu.VMEM((nch16, L), jnp.int32),
            pltpu.VMEM((N_ACC,), jnp.float32),
        ],
        compiler_params=pltpu.CompilerParams(needs_layout_passes=False),
    )


DEPTH = 8


def _prop_body(nchunk, hs_hbm, src_hbm, dst_hbm, out_hbm,
               src_v, dst_v, bufs, accum, gsems, ssems):
    c = lax.axis_index("c")
    s = lax.axis_index("s")
    wid = c * NS + s

    pltpu.sync_copy(src_hbm.at[wid], src_v)
    pltpu.sync_copy(dst_hbm.at[wid], dst_v)

    zero16 = jnp.zeros((L,), jnp.float32)
    zrow = bufs.at[0]

    @pl.loop(0, CHUNK)
    def _(r):
        for q in range(HID_DIM // L):
            zrow[r, pl.ds(q * L, L)] = zero16

    base = s * ROWS_PER_TILE
    nfull = ROWS_PER_TILE // CHUNK
    rem = ROWS_PER_TILE - nfull * CHUNK
    for p in range(nfull):
        pltpu.sync_copy(zrow, accum.at[pl.ds(base + p * CHUNK, CHUNK)])
    if rem:
        pltpu.sync_copy(zrow.at[pl.ds(0, rem)],
                        accum.at[pl.ds(base + nfull * CHUNK, rem)])

    plsc.subcore_barrier()

    @pl.loop(0, nchunk // DEPTH)
    def _(t):
        j0 = DEPTH * t
        gd = [pltpu.async_copy(hs_hbm.at[src_v.at[j0 + k]], bufs.at[k],
                               gsems.at[k])
              for k in range(DEPTH)]
        sd = []
        for k in range(DEPTH):
            gd[k].wait()
            sd.append(pltpu.async_copy(bufs.at[k],
                                       accum.at[dst_v.at[j0 + k]],
                                       ssems.at[k], add=True))
        for k in range(DEPTH):
            sd[k].wait()

    plsc.subcore_barrier()

    pltpu.sync_copy(accum.at[pl.ds(base, ROWS_PER_TILE)],
                    out_hbm.at[c, pl.ds(base, ROWS_PER_TILE)])


def _make_prop_call(nchunk):
    mesh = plsc.VectorSubcoreMesh(core_axis_name="c", subcore_axis_name="s")
    return pl.kernel(
        functools.partial(_prop_body, nchunk),
        out_type=jax.ShapeDtypeStruct((NC, N_ACC, HID_DIM), jnp.float32),
        mesh=mesh,
        scratch_types=[
            pltpu.VMEM((nchunk, CHUNK), jnp.int32),
            pltpu.VMEM((nchunk, CHUNK), jnp.int32),
            pltpu.VMEM((DEPTH, CHUNK, HID_DIM), jnp.float32),
            pltpu.VMEM_SHARED((N_ACC, HID_DIM), jnp.float32),
            pltpu.SemaphoreType.DMA((DEPTH,)),
            pltpu.SemaphoreType.DMA((DEPTH,)),
        ],
        compiler_params=pltpu.CompilerParams(use_tc_tiling_on_sc=False),
    )


ROW_BLK = 2000
GRID = N // ROW_BLK


def _dinv_body(de_ref, do_ref, dinv_ref):
    e = lax.rsqrt(jnp.sum(de_ref[...], axis=1, keepdims=True) + 1.0)
    o = lax.rsqrt(jnp.sum(do_ref[...], axis=1, keepdims=True) + 1.0)
    lanes = lax.broadcasted_iota(jnp.int32, (N_ACC // 2, 2 * HID_DIM), 1)
    dinv_ref[...] = jnp.where(lanes < HID_DIM, e, o)


_dinv_call = pl.pallas_call(
    _dinv_body,
    out_shape=jax.ShapeDtypeStruct((N_ACC // 2, 2 * HID_DIM), jnp.float32),
)


def _m1_body(xp_ref, w1bd_ref, dinv_ref, hs_ref):
    hs_ref[...] = dinv_ref[...] * jnp.dot(xp_ref[...], w1bd_ref[...],
                                          preferred_element_type=jnp.float32)


def _m2_body(s1_ref, hs1_ref, dinv_ref, w2bd_ref, b1_ref, hs2_ref):
    dinv = dinv_ref[...]
    h1 = dinv * (s1_ref[0] + s1_ref[1] + hs1_ref[...]) + b1_ref[...]
    hs2_ref[...] = dinv * jnp.dot(h1, w2bd_ref[...],
                                  preferred_element_type=jnp.float32)


def _m3_body(s2_ref, hs2_ref, dinv_ref, b2_ref, out_ref):
    out_ref[...] = (dinv_ref[...] * (s2_ref[0] + s2_ref[1] + hs2_ref[...])
                    + b2_ref[...])


HB = ROW_BLK // 2
PK = 2 * HID_DIM

_m1_call = pl.pallas_call(
    _m1_body,
    grid=(GRID,),
    in_specs=[
        pl.BlockSpec((HB, 2 * IN_DIM), lambda i: (i, 0)),
        pl.BlockSpec((2 * IN_DIM, PK), lambda i: (0, 0)),
        pl.BlockSpec((HB, PK), lambda i: (i, 0)),
    ],
    out_specs=pl.BlockSpec((HB, PK), lambda i: (i, 0)),
    out_shape=jax.ShapeDtypeStruct((N_ACC // 2, PK), jnp.float32),
)

_m2_call = pl.pallas_call(
    _m2_body,
    grid=(GRID,),
    in_specs=[
        pl.BlockSpec((NC, HB, PK), lambda i: (0, i, 0)),
        pl.BlockSpec((HB, PK), lambda i: (i, 0)),
        pl.BlockSpec((HB, PK), lambda i: (i, 0)),
        pl.BlockSpec((PK, PK), lambda i: (0, 0)),
        pl.BlockSpec((1, PK), lambda i: (0, 0)),
    ],
    out_specs=pl.BlockSpec((HB, PK), lambda i: (i, 0)),
    out_shape=jax.ShapeDtypeStruct((N_ACC // 2, PK), jnp.float32),
)

_m3_call = pl.pallas_call(
    _m3_body,
    grid=(GRID,),
    in_specs=[
        pl.BlockSpec((NC, HB, PK), lambda i: (0, i, 0)),
        pl.BlockSpec((HB, PK), lambda i: (i, 0)),
        pl.BlockSpec((HB, PK), lambda i: (i, 0)),
        pl.BlockSpec((1, PK), lambda i: (0, 0)),
    ],
    out_specs=pl.BlockSpec((HB, PK), lambda i: (i, 0)),
    out_shape=jax.ShapeDtypeStruct((N // 2, PK), jnp.float32),
)


@jax.jit
def kernel(x, edge_index, W1, b1, W2, b2):
    E = edge_index.shape[1]
    grain = NW * CHUNK * DEPTH
    e_pad = ((E + grain - 1) // grain) * grain
    nchunk = e_pad // (NW * CHUNK)
    pad = e_pad - E

    src = edge_index[0]
    dst = edge_index[1]
    ar = jnp.arange(pad, dtype=dst.dtype)
    dump = N + jax.lax.rem(ar, jnp.asarray(N_ACC - N, dst.dtype))
    fake_src = jax.lax.rem(ar * 257, jnp.asarray(N, src.dtype))
    src_p = jnp.concatenate([src, fake_src])
    dst_p = jnp.concatenate([dst, dump])
    src_g = src_p.reshape(NW, nchunk, CHUNK)
    dst_g = dst_p.reshape(NW, nchunk, CHUNK)
    dst_d = dst_p.reshape(NW, e_pad // (NW * L), L)

    degp = _make_deg_call(e_pad // (NW * L))(dst_d)
    dinvp = _dinv_call(degp[:, 0::2].T, degp[:, 1::2].T)
    prop = _make_prop_call(nchunk)

    xp = x.reshape(N // 2, 2 * IN_DIM)
    zji = jnp.zeros((IN_DIM, HID_DIM), jnp.float32)
    w1bd = jnp.block([[W1, zji], [zji, W1]])
    zjj = jnp.zeros((HID_DIM, HID_DIM), jnp.float32)
    w2bd = jnp.block([[W2, zjj], [zjj, W2]])
    b1p = jnp.concatenate([b1, b1]).reshape(1, PK)
    b2p = jnp.concatenate([b2, b2]).reshape(1, PK)

    hs1p = _m1_call(xp, w1bd, dinvp)
    s1 = prop(hs1p.reshape(N_ACC, HID_DIM), src_g, dst_g)
    s1p = s1.reshape(NC, N_ACC // 2, PK)
    hs2p = _m2_call(s1p, hs1p, dinvp, w2bd, b1p)
    s2 = prop(hs2p.reshape(N_ACC, HID_DIM), src_g, dst_g)
    s2p = s2.reshape(NC, N_ACC // 2, PK)
    outp = _m3_call(s2p, hs2p, dinvp, b2p)
    return outp.reshape(N, HID_DIM)


# speedup vs baseline: 1.1382x; 1.1382x over previous
import functools

import jax
import jax.numpy as jnp
from jax import lax
from jax.experimental import pallas as pl
from jax.experimental.pallas import tpu as pltpu
from jax.experimental.pallas import tpu_sc as plsc

N = 10000
IN_DIM = 128
HID_DIM = 64

NC = 2
NS = 16
NW = NC * NS
L = 16

CHUNK = 128
N_ACC = 10112
ROWS_PER_TILE = N_ACC // NS


def _flat_tile_id():
    return lax.axis_index("c") * NS + lax.axis_index("s")


def _deg_body(nch16, dst_hbm, out_hbm, dst_v, acc):
    wid = _flat_tile_id()
    pltpu.sync_copy(dst_hbm.at[wid], dst_v)

    zero16 = jnp.zeros((L,), jnp.float32)

    @pl.loop(0, N_ACC // L)
    def _(i):
        acc[pl.ds(i * L, L)] = zero16

    ones16 = jnp.ones((L,), jnp.float32)

    @pl.loop(0, nch16)
    def _(k):
        idx = dst_v[k]
        plsc.addupdate_scatter(acc, [idx], ones16)

    pltpu.sync_copy(acc, out_hbm.at[wid])


def _make_deg_call(nch16):
    mesh = plsc.VectorSubcoreMesh(core_axis_name="c", subcore_axis_name="s")
    return pl.kernel(
        functools.partial(_deg_body, nch16),
        out_type=jax.ShapeDtypeStruct((NW, N_ACC), jnp.float32),
        mesh=mesh,
        scratch_types=[
            pltpu.VMEM((nch16, L), jnp.int32),
            pltpu.VMEM((N_ACC,), jnp.float32),
        ],
        compiler_params=pltpu.CompilerParams(needs_layout_passes=False),
    )


DEPTH = 8


def _prop_body(nchunk, hs_hbm, src_hbm, dst_hbm, out_hbm,
               src_v, dst_v, bufs, accum, gsems, ssems):
    c = lax.axis_index("c")
    s = lax.axis_index("s")
    wid = c * NS + s

    pltpu.sync_copy(src_hbm.at[wid], src_v)
    pltpu.sync_copy(dst_hbm.at[wid], dst_v)

    zero16 = jnp.zeros((L,), jnp.float32)
    zrow = bufs.at[0]

    @pl.loop(0, CHUNK)
    def _(r):
        for q in range(HID_DIM // L):
            zrow[r, pl.ds(q * L, L)] = zero16

    base = s * ROWS_PER_TILE
    nfull = ROWS_PER_TILE // CHUNK
    rem = ROWS_PER_TILE - nfull * CHUNK
    for p in range(nfull):
        pltpu.sync_copy(zrow, accum.at[pl.ds(base + p * CHUNK, CHUNK)])
    if rem:
        pltpu.sync_copy(zrow.at[pl.ds(0, rem)],
                        accum.at[pl.ds(base + nfull * CHUNK, rem)])

    plsc.subcore_barrier()

    @pl.loop(0, nchunk // DEPTH)
    def _(t):
        j0 = DEPTH * t
        gd = [pltpu.async_copy(hs_hbm.at[src_v.at[j0 + k]], bufs.at[k],
                               gsems.at[k])
              for k in range(DEPTH)]
        sd = []
        for k in range(DEPTH):
            gd[k].wait()
            sd.append(pltpu.async_copy(bufs.at[k],
                                       accum.at[dst_v.at[j0 + k]],
                                       ssems.at[k], add=True))
        for k in range(DEPTH):
            sd[k].wait()

    plsc.subcore_barrier()

    pltpu.sync_copy(accum.at[pl.ds(base, ROWS_PER_TILE)],
                    out_hbm.at[c, pl.ds(base, ROWS_PER_TILE)])


def _make_prop_call(nchunk):
    mesh = plsc.VectorSubcoreMesh(core_axis_name="c", subcore_axis_name="s")
    return pl.kernel(
        functools.partial(_prop_body, nchunk),
        out_type=jax.ShapeDtypeStruct((NC, N_ACC, HID_DIM), jnp.float32),
        mesh=mesh,
        scratch_types=[
            pltpu.VMEM((nchunk, CHUNK), jnp.int32),
            pltpu.VMEM((nchunk, CHUNK), jnp.int32),
            pltpu.VMEM((DEPTH, CHUNK, HID_DIM), jnp.float32),
            pltpu.VMEM_SHARED((N_ACC, HID_DIM), jnp.float32),
            pltpu.SemaphoreType.DMA((DEPTH,)),
            pltpu.SemaphoreType.DMA((DEPTH,)),
        ],
        compiler_params=pltpu.CompilerParams(use_tc_tiling_on_sc=False),
    )


ROW_BLK = 2000
GRID = N // ROW_BLK


def _dinv_body(degp_ref, ones_ref, dinv_ref):
    deg_col = lax.dot_general(degp_ref[...], ones_ref[...],
                              (((0,), (0,)), ((), ())),
                              precision=lax.Precision.HIGHEST,
                              preferred_element_type=jnp.float32)
    e = lax.rsqrt(deg_col[:N_ACC // 2] + 1.0)
    o = lax.rsqrt(deg_col[N_ACC // 2:] + 1.0)
    lanes = lax.broadcasted_iota(jnp.int32, (N_ACC // 2, 2 * HID_DIM), 1)
    dinv_ref[...] = jnp.where(lanes < HID_DIM, e, o)


_dinv_call = pl.pallas_call(
    _dinv_body,
    out_shape=jax.ShapeDtypeStruct((N_ACC // 2, 2 * HID_DIM), jnp.float32),
)


def _m1_body(xp_ref, w1bd_ref, dinv_ref, hs_ref):
    hs_ref[...] = dinv_ref[...] * jnp.dot(xp_ref[...], w1bd_ref[...],
                                          preferred_element_type=jnp.float32)


def _m2_body(s1_ref, hs1_ref, dinv_ref, w2bd_ref, b1_ref, hs2_ref):
    dinv = dinv_ref[...]
    h1 = dinv * (s1_ref[0] + s1_ref[1] + hs1_ref[...]) + b1_ref[...]
    hs2_ref[...] = dinv * jnp.dot(h1, w2bd_ref[...],
                                  preferred_element_type=jnp.float32)


def _m3_body(s2_ref, hs2_ref, dinv_ref, b2_ref, out_ref):
    out_ref[...] = (dinv_ref[...] * (s2_ref[0] + s2_ref[1] + hs2_ref[...])
                    + b2_ref[...])


HB = ROW_BLK // 2
PK = 2 * HID_DIM

_m1_call = pl.pallas_call(
    _m1_body,
    grid=(GRID,),
    in_specs=[
        pl.BlockSpec((HB, 2 * IN_DIM), lambda i: (i, 0)),
        pl.BlockSpec((2 * IN_DIM, PK), lambda i: (0, 0)),
        pl.BlockSpec((HB, PK), lambda i: (i, 0)),
    ],
    out_specs=pl.BlockSpec((HB, PK), lambda i: (i, 0)),
    out_shape=jax.ShapeDtypeStruct((N_ACC // 2, PK), jnp.float32),
)

_m2_call = pl.pallas_call(
    _m2_body,
    grid=(GRID,),
    in_specs=[
        pl.BlockSpec((NC, HB, PK), lambda i: (0, i, 0)),
        pl.BlockSpec((HB, PK), lambda i: (i, 0)),
        pl.BlockSpec((HB, PK), lambda i: (i, 0)),
        pl.BlockSpec((PK, PK), lambda i: (0, 0)),
        pl.BlockSpec((1, PK), lambda i: (0, 0)),
    ],
    out_specs=pl.BlockSpec((HB, PK), lambda i: (i, 0)),
    out_shape=jax.ShapeDtypeStruct((N_ACC // 2, PK), jnp.float32),
)

_m3_call = pl.pallas_call(
    _m3_body,
    grid=(GRID,),
    in_specs=[
        pl.BlockSpec((NC, HB, PK), lambda i: (0, i, 0)),
        pl.BlockSpec((HB, PK), lambda i: (i, 0)),
        pl.BlockSpec((HB, PK), lambda i: (i, 0)),
        pl.BlockSpec((1, PK), lambda i: (0, 0)),
    ],
    out_specs=pl.BlockSpec((HB, PK), lambda i: (i, 0)),
    out_shape=jax.ShapeDtypeStruct((N // 2, PK), jnp.float32),
)


@jax.jit
def kernel(x, edge_index, W1, b1, W2, b2):
    E = edge_index.shape[1]
    grain = NW * CHUNK * DEPTH
    e_pad = ((E + grain - 1) // grain) * grain
    nchunk = e_pad // (NW * CHUNK)
    pad = e_pad - E

    src = edge_index[0]
    dst = edge_index[1]
    ar = jnp.arange(pad, dtype=dst.dtype)
    dump = N + jax.lax.rem(ar, jnp.asarray(N_ACC - N, dst.dtype))
    fake_src = jax.lax.rem(ar * 257, jnp.asarray(N, src.dtype))
    src_p = jnp.concatenate([src, fake_src])
    dst_p = jnp.concatenate([dst, dump])
    src_g = src_p.reshape(NW, nchunk, CHUNK)
    dst_g = dst_p.reshape(NW, nchunk, CHUNK)
    dst_k = (dst_p % 2) * (N_ACC // 2) + dst_p // 2
    dst_d = dst_k.reshape(NW, e_pad // (NW * L), L)

    degp = _make_deg_call(e_pad // (NW * L))(dst_d)
    dinvp = _dinv_call(degp, jnp.ones((NW, 1), jnp.float32))
    prop = _make_prop_call(nchunk)

    xp = x.reshape(N // 2, 2 * IN_DIM)
    zji = jnp.zeros((IN_DIM, HID_DIM), jnp.float32)
    w1bd = jnp.block([[W1, zji], [zji, W1]])
    zjj = jnp.zeros((HID_DIM, HID_DIM), jnp.float32)
    w2bd = jnp.block([[W2, zjj], [zjj, W2]])
    b1p = jnp.concatenate([b1, b1]).reshape(1, PK)
    b2p = jnp.concatenate([b2, b2]).reshape(1, PK)

    hs1p = _m1_call(xp, w1bd, dinvp)
    s1 = prop(hs1p.reshape(N_ACC, HID_DIM), src_g, dst_g)
    s1p = s1.reshape(NC, N_ACC // 2, PK)
    hs2p = _m2_call(s1p, hs1p, dinvp, w2bd, b1p)
    s2 = prop(hs2p.reshape(N_ACC, HID_DIM), src_g, dst_g)
    s2p = s2.reshape(NC, N_ACC // 2, PK)
    outp = _m3_call(s2p, hs2p, dinvp, b2p)
    return outp.reshape(N, HID_DIM)


# speedup vs baseline: 1.2175x; 1.0697x over previous
import functools

import jax
import jax.numpy as jnp
from jax import lax
from jax.experimental import pallas as pl
from jax.experimental.pallas import tpu as pltpu
from jax.experimental.pallas import tpu_sc as plsc

N = 10000
IN_DIM = 128
HID_DIM = 64

NC = 2
NS = 16
NW = NC * NS
L = 16

CHUNK = 128
N_ACC = 10112
ROWS_PER_TILE = N_ACC // NS


def _flat_tile_id():
    return lax.axis_index("c") * NS + lax.axis_index("s")


def _deg_body(nch16, dst_hbm, out_hbm, dst_v, acc):
    wid = _flat_tile_id()
    pltpu.sync_copy(dst_hbm.at[wid], dst_v)

    zero16 = jnp.zeros((L,), jnp.float32)

    @pl.loop(0, N_ACC // L)
    def _(i):
        acc[pl.ds(i * L, L)] = zero16

    ones16 = jnp.ones((L,), jnp.float32)

    @pl.loop(0, nch16)
    def _(k):
        idx = dst_v[k]
        plsc.addupdate_scatter(acc, [idx], ones16)

    pltpu.sync_copy(acc, out_hbm.at[wid])


def _make_deg_call(nch16):
    mesh = plsc.VectorSubcoreMesh(core_axis_name="c", subcore_axis_name="s")
    return pl.kernel(
        functools.partial(_deg_body, nch16),
        out_type=jax.ShapeDtypeStruct((NW, N_ACC), jnp.float32),
        mesh=mesh,
        scratch_types=[
            pltpu.VMEM((nch16, L), jnp.int32),
            pltpu.VMEM((N_ACC,), jnp.float32),
        ],
        compiler_params=pltpu.CompilerParams(needs_layout_passes=False),
    )


DEPTH = 8


def _prop_body(nchunk, hs_hbm, src_hbm, dst_hbm, out_hbm,
               src_v, dst_v, bufs, accum, gsems, ssems):
    c = lax.axis_index("c")
    s = lax.axis_index("s")
    wid = c * NS + s

    pltpu.sync_copy(src_hbm.at[wid], src_v)
    pltpu.sync_copy(dst_hbm.at[wid], dst_v)

    zero16 = jnp.zeros((L,), jnp.float32)
    zrow = bufs.at[0]

    @pl.loop(0, CHUNK)
    def _(r):
        for q in range(HID_DIM // L):
            zrow[r, pl.ds(q * L, L)] = zero16

    base = s * ROWS_PER_TILE
    nfull = ROWS_PER_TILE // CHUNK
    rem = ROWS_PER_TILE - nfull * CHUNK
    for p in range(nfull):
        pltpu.sync_copy(zrow, accum.at[pl.ds(base + p * CHUNK, CHUNK)])
    if rem:
        pltpu.sync_copy(zrow.at[pl.ds(0, rem)],
                        accum.at[pl.ds(base + nfull * CHUNK, rem)])

    plsc.subcore_barrier()

    @pl.loop(0, nchunk // DEPTH)
    def _(t):
        j0 = DEPTH * t
        gd = [pltpu.async_copy(hs_hbm.at[src_v.at[j0 + k]], bufs.at[k],
                               gsems.at[k])
              for k in range(DEPTH)]
        sd = []
        for k in range(DEPTH):
            gd[k].wait()
            sd.append(pltpu.async_copy(bufs.at[k],
                                       accum.at[dst_v.at[j0 + k]],
                                       ssems.at[k], add=True))
        for k in range(DEPTH):
            sd[k].wait()

    plsc.subcore_barrier()

    pltpu.sync_copy(accum.at[pl.ds(base, ROWS_PER_TILE)],
                    out_hbm.at[c, pl.ds(base, ROWS_PER_TILE)])


def _make_prop_call(nchunk):
    mesh = plsc.VectorSubcoreMesh(core_axis_name="c", subcore_axis_name="s")
    return pl.kernel(
        functools.partial(_prop_body, nchunk),
        out_type=jax.ShapeDtypeStruct((NC, N_ACC, HID_DIM), jnp.float32),
        mesh=mesh,
        scratch_types=[
            pltpu.VMEM((nchunk, CHUNK), jnp.int32),
            pltpu.VMEM((nchunk, CHUNK), jnp.int32),
            pltpu.VMEM((DEPTH, CHUNK, HID_DIM), jnp.float32),
            pltpu.VMEM_SHARED((N_ACC, HID_DIM), jnp.float32),
            pltpu.SemaphoreType.DMA((DEPTH,)),
            pltpu.SemaphoreType.DMA((DEPTH,)),
        ],
        compiler_params=pltpu.CompilerParams(use_tc_tiling_on_sc=False),
    )


ROW_BLK = 2000
GRID = N // ROW_BLK


def _dinv_body(degp_ref, ones_ref, dinv_ref):
    deg_col = lax.dot_general(degp_ref[...], ones_ref[...],
                              (((0,), (0,)), ((), ())),
                              precision=lax.Precision.HIGHEST,
                              preferred_element_type=jnp.float32)
    e = lax.rsqrt(deg_col[:N_ACC // 2] + 1.0)
    o = lax.rsqrt(deg_col[N_ACC // 2:] + 1.0)
    lanes = lax.broadcasted_iota(jnp.int32, (N_ACC // 2, 2 * HID_DIM), 1)
    dinv_ref[...] = jnp.where(lanes < HID_DIM, e, o)


_dinv_call = pl.pallas_call(
    _dinv_body,
    out_shape=jax.ShapeDtypeStruct((N_ACC // 2, 2 * HID_DIM), jnp.float32),
)


def _m1_body(xp_ref, w1bd_ref, dinv_ref, hs_ref):
    hs_ref[...] = dinv_ref[...] * jnp.dot(xp_ref[...], w1bd_ref[...],
                                          preferred_element_type=jnp.float32)


def _m2_body(s1_ref, hs1_ref, dinv_ref, w2bd_ref, b1_ref, hs2_ref):
    dinv = dinv_ref[...]
    h1 = dinv * (s1_ref[0] + s1_ref[1] + hs1_ref[...]) + b1_ref[...]
    hs2_ref[...] = dinv * jnp.dot(h1, w2bd_ref[...],
                                  preferred_element_type=jnp.float32)


def _m3_body(s2_ref, hs2_ref, dinv_ref, b2_ref, out_ref):
    out_ref[...] = (dinv_ref[...] * (s2_ref[0] + s2_ref[1] + hs2_ref[...])
                    + b2_ref[...])


HB = ROW_BLK // 2
PK = 2 * HID_DIM

_m1_call = pl.pallas_call(
    _m1_body,
    grid=(GRID,),
    in_specs=[
        pl.BlockSpec((HB, 2 * IN_DIM), lambda i: (i, 0)),
        pl.BlockSpec((2 * IN_DIM, PK), lambda i: (0, 0)),
        pl.BlockSpec((HB, PK), lambda i: (i, 0)),
    ],
    out_specs=pl.BlockSpec((HB, PK), lambda i: (i, 0)),
    out_shape=jax.ShapeDtypeStruct((N_ACC // 2, PK), jnp.float32),
)

_m2_call = pl.pallas_call(
    _m2_body,
    grid=(GRID,),
    in_specs=[
        pl.BlockSpec((NC, HB, PK), lambda i: (0, i, 0)),
        pl.BlockSpec((HB, PK), lambda i: (i, 0)),
        pl.BlockSpec((HB, PK), lambda i: (i, 0)),
        pl.BlockSpec((PK, PK), lambda i: (0, 0)),
        pl.BlockSpec((1, PK), lambda i: (0, 0)),
    ],
    out_specs=pl.BlockSpec((HB, PK), lambda i: (i, 0)),
    out_shape=jax.ShapeDtypeStruct((N_ACC // 2, PK), jnp.float32),
)

_m3_call = pl.pallas_call(
    _m3_body,
    grid=(GRID,),
    in_specs=[
        pl.BlockSpec((NC, HB, PK), lambda i: (0, i, 0)),
        pl.BlockSpec((HB, PK), lambda i: (i, 0)),
        pl.BlockSpec((HB, PK), lambda i: (i, 0)),
        pl.BlockSpec((1, PK), lambda i: (0, 0)),
    ],
    out_specs=pl.BlockSpec((HB, PK), lambda i: (i, 0)),
    out_shape=jax.ShapeDtypeStruct((N // 2, PK), jnp.float32),
)


@jax.jit
def kernel(x, edge_index, W1, b1, W2, b2):
    E = edge_index.shape[1]
    grain = NW * CHUNK * DEPTH
    e_pad = ((E + grain - 1) // grain) * grain
    nchunk = e_pad // (NW * CHUNK)
    pad = e_pad - E

    ei = edge_index.reshape(2, E // CHUNK, CHUNK)
    src = ei[0].reshape(E)
    dst = ei[1].reshape(E)
    ar = jnp.arange(pad, dtype=dst.dtype)
    dump = N + jax.lax.rem(ar, jnp.asarray(N_ACC - N, dst.dtype))
    fake_src = jax.lax.rem(ar * 257, jnp.asarray(N, src.dtype))
    src_p = jnp.concatenate([src, fake_src])
    dst_p = jnp.concatenate([dst, dump])
    src_g = src_p.reshape(NW, nchunk, CHUNK)
    dst_g = dst_p.reshape(NW, nchunk, CHUNK)
    dst_k = (dst_p & 1) * (N_ACC // 2) + (dst_p >> 1)
    dst_d = dst_k.reshape(NW, e_pad // (NW * L), L)

    degp = _make_deg_call(e_pad // (NW * L))(dst_d)
    dinvp = _dinv_call(degp, jnp.ones((NW, 1), jnp.float32))
    prop = _make_prop_call(nchunk)

    xp = x.reshape(N // 2, 2 * IN_DIM)
    zji = jnp.zeros((IN_DIM, HID_DIM), jnp.float32)
    w1bd = jnp.block([[W1, zji], [zji, W1]])
    zjj = jnp.zeros((HID_DIM, HID_DIM), jnp.float32)
    w2bd = jnp.block([[W2, zjj], [zjj, W2]])
    b1p = jnp.concatenate([b1, b1]).reshape(1, PK)
    b2p = jnp.concatenate([b2, b2]).reshape(1, PK)

    hs1p = _m1_call(xp, w1bd, dinvp)
    s1 = prop(hs1p.reshape(N_ACC, HID_DIM), src_g, dst_g)
    s1p = s1.reshape(NC, N_ACC // 2, PK)
    hs2p = _m2_call(s1p, hs1p, dinvp, w2bd, b1p)
    s2 = prop(hs2p.reshape(N_ACC, HID_DIM), src_g, dst_g)
    s2p = s2.reshape(NC, N_ACC // 2, PK)
    outp = _m3_call(s2p, hs2p, dinvp, b2p)
    return outp.reshape(N, HID_DIM)
